# baseline (speedup 1.0000x reference)
import functools
import math

import jax
import jax.numpy as jnp
import numpy as np
from jax import lax
from jax.experimental import pallas as pl
from jax.experimental.pallas import tpu as pltpu
from jax.experimental.pallas import tpu_sc as plsc

_LANES = 16


def _pe_table(rows, d):
    position = np.arange(rows, dtype=np.float32)[:, None]
    div_term = np.exp(np.arange(0, d, 2, dtype=np.float32) * (-(math.log(10000.0) / d)))
    pe = np.zeros((rows, d), dtype=np.float32)
    pe[:, 0::2] = np.sin(position * div_term)
    pe[:, 1::2] = np.cos(position * div_term)
    return pe


@functools.lru_cache(maxsize=None)
def _build(seq, batch, vocab, d, tw):
    info = plsc.get_sparse_core_info()
    nc, ns = info.num_cores, info.num_subcores
    nw = nc * ns
    n = seq * batch
    K = 256
    NSUB = K // 128
    TB = K // 128
    ET = d // 8
    BT = batch // 128
    T = n // K
    PW = T // nw
    CPS = batch // K
    assert n % K == 0 and T % nw == 0 and batch % K == 0 and PW % 2 == 0
    scale = math.sqrt(d)
    nq = d // _LANES
    pe_span = PW // CPS + 2

    mesh = plsc.VectorSubcoreMesh(core_axis_name="c", subcore_axis_name="s")

    @functools.partial(
        pl.kernel,
        out_type=jax.ShapeDtypeStruct((seq * ET * BT, 1024), jnp.float32),
        mesh=mesh,
        scratch_types=[
            pltpu.VMEM((NSUB, 128), jnp.int32),
            pltpu.VMEM((NSUB, 128), jnp.int32),
            pltpu.VMEM((K, tw), jnp.float32),
            pltpu.VMEM((K, tw), jnp.float32),
            pltpu.VMEM((ET, TB, 1024), jnp.float32),
            pltpu.VMEM((ET, TB, 1024), jnp.float32),
            pltpu.VMEM((pe_span, d), jnp.float32),
            pltpu.SemaphoreType.DMA,
            pltpu.SemaphoreType.DMA,
            pltpu.SemaphoreType.DMA,
        ],
        compiler_params=pltpu.CompilerParams(use_tc_tiling_on_sc=False),
    )
    def emb(idx_hbm, table_hbm, pe_hbm, out_hbm,
            idx0_v, idx1_v, rows0_v, rows1_v, tr0_v, tr1_v, pe_v,
            g_sem0, g_sem1, o_sem):
        idx_bufs = (idx0_v, idx1_v)
        rows_bufs = (rows0_v, rows1_v)
        tr_bufs = (tr0_v, tr1_v)
        g_sems = (g_sem0, g_sem1)
        wid = lax.axis_index("s") * nc + lax.axis_index("c")
        base = wid * PW
        spos0 = base // CPS

        def sync_idx(t, b):
            pltpu.sync_copy(idx_hbm.at[pl.ds((base + t) * NSUB, NSUB)], idx_bufs[b])

        def gather_copies(b):
            return [
                pltpu.make_async_copy(
                    table_hbm.at[idx_bufs[b].at[jj]],
                    rows_bufs[b].at[pl.ds(jj * 128, 128)],
                    g_sems[b],
                )
                for jj in range(NSUB)
            ]

        def out_copies(t, b):
            g = base + t
            s = g // CPS
            c = g % CPS
            return [
                pltpu.make_async_copy(
                    tr_bufs[b].at[et],
                    out_hbm.at[pl.ds((s * ET + et) * BT + TB * c, TB)],
                    o_sem,
                )
                for et in range(ET)
            ]

        riota = jax.lax.iota(jnp.int32, _LANES)
        perms = {st: riota ^ st for st in (8, 4, 2, 1)}
        masks = {st: (riota & st) == 0 for st in (8, 4, 2, 1)}

        def compute(t, b):
            g = base + t
            spos_l = g // CPS - spos0
            rows = rows_bufs[b]
            tr = tr_bufs[b]
            pe_regs = [pe_v[spos_l, pl.ds(q * _LANES, _LANES)] for q in range(nq)]

            def kb_body(kb, kc):
                b0 = kb * _LANES
                btl = lax.shift_right_logical(kb, 3)
                bi0 = lax.bitwise_and(kb, 7) * _LANES
                for q in range(nq):
                    x = [
                        rows[b0 + i, pl.ds(q * _LANES, _LANES)] * scale + pe_regs[q]
                        for i in range(_LANES)
                    ]
                    for st in (8, 4, 2, 1):
                        y = list(x)
                        for i in range(_LANES):
                            if i & st:
                                continue
                            j = i ^ st
                            u = jnp.where(masks[st], x[j], x[i])
                            up = u.at[perms[st]].get(mode="promise_in_bounds")
                            y[i] = jnp.where(masks[st], x[i], up)
                            y[j] = jnp.where(masks[st], up, x[j])
                        x = y
                    for j in range(_LANES):
                        e = q * _LANES + j
                        tr[e // 8, btl, pl.ds((e % 8) * 128 + bi0, _LANES)] = x[j]
                return kc

            lax.fori_loop(0, K // _LANES, kb_body, 0)

        pltpu.sync_copy(pe_hbm.at[pl.ds(spos0, pe_span)], pe_v)
        sync_idx(0, 0)
        for cp in gather_copies(0):
            cp.start()
        sync_idx(1, 1)

        def outer(i, carry):
            for b in range(2):
                t = 2 * i + b
                if b == 1:
                    for cp in out_copies(t - 1, 1 - b):
                        cp.wait()
                else:
                    @pl.when(i >= 1)
                    def _():
                        for cp in out_copies(t - 1, 1 - b):
                            cp.wait()
                @pl.when(2 * i + b + 1 <= PW - 1)
                def _():
                    for cp in gather_copies(1 - b):
                        cp.start()
                for cp in gather_copies(b):
                    cp.wait()
                @pl.when(2 * i + b + 2 <= PW - 1)
                def _():
                    sync_idx(t + 2, b)
                compute(t, b)
                for cp in out_copies(t, b):
                    cp.start()
            return carry

        lax.fori_loop(0, PW // 2, outer, 0)
        for cp in out_copies(PW - 1, 1):
            cp.wait()

    return emb


def kernel(batch, embed_weight):
    seq, bsz = batch.shape
    vocab, d = embed_weight.shape
    idx = batch.astype(jnp.int32).reshape(seq * bsz // 128, 128)
    pw = (seq * bsz // 256) // 32
    pe_rows = seq + pw // (bsz // 256) + 2
    pe = jnp.asarray(_pe_table(pe_rows, d))
    z = _build(seq, bsz, vocab, d, d)(idx, embed_weight, pe)
    z5 = z.reshape(seq, d // 8, bsz // 128, 8, 128)
    out = jnp.transpose(z5, (0, 2, 4, 1, 3)).reshape(seq, bsz, d)
    return out


# speedup vs baseline: 1.0002x; 1.0002x over previous
import functools
import math

import jax
import jax.numpy as jnp
import numpy as np
from jax import lax
from jax.experimental import pallas as pl
from jax.experimental.pallas import tpu as pltpu
from jax.experimental.pallas import tpu_sc as plsc

_LANES = 16


def _pe_table(rows, d):
    position = np.arange(rows, dtype=np.float32)[:, None]
    div_term = np.exp(np.arange(0, d, 2, dtype=np.float32) * (-(math.log(10000.0) / d)))
    pe = np.zeros((rows, d), dtype=np.float32)
    pe[:, 0::2] = np.sin(position * div_term)
    pe[:, 1::2] = np.cos(position * div_term)
    return pe


@functools.lru_cache(maxsize=None)
def _build(seq, batch, vocab, d, tw):
    info = plsc.get_sparse_core_info()
    nc, ns = info.num_cores, info.num_subcores
    nw = nc * ns
    n = seq * batch
    K = 256
    NSUB = K // 128
    TB = K // 128
    ET = d // 8
    BT = batch // 128
    T = n // K
    PW = T // nw
    CPS = batch // K
    assert n % K == 0 and T % nw == 0 and batch % K == 0 and PW % 2 == 0
    scale = math.sqrt(d)
    nq = d // _LANES
    pe_span = PW // CPS + 2

    mesh = plsc.VectorSubcoreMesh(core_axis_name="c", subcore_axis_name="s")

    @functools.partial(
        pl.kernel,
        out_type=jax.ShapeDtypeStruct((seq * ET * BT, 1024), jnp.float32),
        mesh=mesh,
        scratch_types=[
            pltpu.VMEM((NSUB, 128), jnp.int32),
            pltpu.VMEM((NSUB, 128), jnp.int32),
            pltpu.VMEM((K, tw), jnp.float32),
            pltpu.VMEM((K, tw), jnp.float32),
            pltpu.VMEM((ET, TB, 1024), jnp.float32),
            pltpu.VMEM((ET, TB, 1024), jnp.float32),
            pltpu.VMEM((pe_span, d), jnp.float32),
            pltpu.SemaphoreType.DMA,
            pltpu.SemaphoreType.DMA,
            pltpu.SemaphoreType.DMA,
        ],
        compiler_params=pltpu.CompilerParams(use_tc_tiling_on_sc=False),
    )
    def emb(idx_hbm, table_hbm, pe_hbm, out_hbm,
            idx0_v, idx1_v, rows0_v, rows1_v, tr0_v, tr1_v, pe_v,
            g_sem0, g_sem1, o_sem):
        idx_bufs = (idx0_v, idx1_v)
        rows_bufs = (rows0_v, rows1_v)
        tr_bufs = (tr0_v, tr1_v)
        g_sems = (g_sem0, g_sem1)
        wid = lax.axis_index("s") * nc + lax.axis_index("c")
        base = wid * PW
        spos0 = base // CPS

        def sync_idx(t, b):
            pltpu.sync_copy(idx_hbm.at[pl.ds((base + t) * NSUB, NSUB)], idx_bufs[b])

        def gather_copies(b):
            return [
                pltpu.make_async_copy(
                    table_hbm.at[idx_bufs[b].at[jj]],
                    rows_bufs[b].at[pl.ds(jj * 128, 128)],
                    g_sems[b],
                )
                for jj in range(NSUB)
            ]

        def out_copies(t, b):
            g = base + t
            s = g // CPS
            c = g % CPS
            return [
                pltpu.make_async_copy(
                    tr_bufs[b].at[et],
                    out_hbm.at[pl.ds((s * ET + et) * BT + TB * c, TB)],
                    o_sem,
                )
                for et in range(ET)
            ]

        riota = jax.lax.iota(jnp.int32, _LANES)
        perms = {st: riota ^ st for st in (8, 4, 2, 1)}
        masks = {st: (riota & st) == 0 for st in (8, 4, 2, 1)}

        def compute(t, b):
            g = base + t
            spos_l = g // CPS - spos0
            rows = rows_bufs[b]
            tr = tr_bufs[b]
            pe_regs = [pe_v[spos_l, pl.ds(q * _LANES, _LANES)] for q in range(nq)]

            def kb_body(kb, kc):
                b0 = kb * _LANES
                btl = lax.shift_right_logical(kb, 3)
                bi0 = lax.bitwise_and(kb, 7) * _LANES
                for q in range(nq):
                    x = [
                        rows[b0 + i, pl.ds(q * _LANES, _LANES)] * scale + pe_regs[q]
                        for i in range(_LANES)
                    ]
                    for st in (8, 4, 2, 1):
                        y = list(x)
                        for i in range(_LANES):
                            if i & st:
                                continue
                            j = i ^ st
                            u = jnp.where(masks[st], x[j], x[i])
                            up = u.at[perms[st]].get(mode="promise_in_bounds")
                            y[i] = jnp.where(masks[st], x[i], up)
                            y[j] = jnp.where(masks[st], up, x[j])
                        x = y
                    for j in range(_LANES):
                        e = q * _LANES + j
                        tr[e // 8, btl, pl.ds((e % 8) * 128 + bi0, _LANES)] = x[j]
                return kc

            lax.fori_loop(0, K // _LANES, kb_body, 0, unroll=2)

        pltpu.sync_copy(pe_hbm.at[pl.ds(spos0, pe_span)], pe_v)
        sync_idx(0, 0)
        for cp in gather_copies(0):
            cp.start()
        sync_idx(1, 1)

        def outer(i, carry):
            for b in range(2):
                t = 2 * i + b
                if b == 1:
                    for cp in out_copies(t - 1, 1 - b):
                        cp.wait()
                else:
                    @pl.when(i >= 1)
                    def _():
                        for cp in out_copies(t - 1, 1 - b):
                            cp.wait()
                @pl.when(2 * i + b + 1 <= PW - 1)
                def _():
                    for cp in gather_copies(1 - b):
                        cp.start()
                for cp in gather_copies(b):
                    cp.wait()
                @pl.when(2 * i + b + 2 <= PW - 1)
                def _():
                    sync_idx(t + 2, b)
                compute(t, b)
                for cp in out_copies(t, b):
                    cp.start()
            return carry

        lax.fori_loop(0, PW // 2, outer, 0)
        for cp in out_copies(PW - 1, 1):
            cp.wait()

    return emb


def kernel(batch, embed_weight):
    seq, bsz = batch.shape
    vocab, d = embed_weight.shape
    idx = batch.astype(jnp.int32).reshape(seq * bsz // 128, 128)
    pw = (seq * bsz // 256) // 32
    pe_rows = seq + pw // (bsz // 256) + 2
    pe = jnp.asarray(_pe_table(pe_rows, d))
    z = _build(seq, bsz, vocab, d, d)(idx, embed_weight, pe)
    z5 = z.reshape(seq, d // 8, bsz // 128, 8, 128)
    out = jnp.transpose(z5, (0, 2, 4, 1, 3)).reshape(seq, bsz, d)
    return out


# speedup vs baseline: 1.0756x; 1.0755x over previous
import functools
import math

import jax
import jax.numpy as jnp
import numpy as np
from jax import lax
from jax.experimental import pallas as pl
from jax.experimental.pallas import tpu as pltpu
from jax.experimental.pallas import tpu_sc as plsc

_LANES = 16


def _pe_table(rows, d):
    position = np.arange(rows, dtype=np.float32)[:, None]
    div_term = np.exp(np.arange(0, d, 2, dtype=np.float32) * (-(math.log(10000.0) / d)))
    pe = np.zeros((rows, d), dtype=np.float32)
    pe[:, 0::2] = np.sin(position * div_term)
    pe[:, 1::2] = np.cos(position * div_term)
    return pe


@functools.lru_cache(maxsize=None)
def _build(seq, batch, vocab, d):
    info = plsc.get_sparse_core_info()
    nc, ns = info.num_cores, info.num_subcores
    nw = nc * ns
    n = seq * batch
    K = 512
    NSUB = K // 128
    TB = K // 128
    ET = d // 8
    BT = batch // 128
    T = n // K
    PW = T // nw
    CPS = batch // K
    assert n % K == 0 and T % nw == 0 and batch % K == 0 and PW % 2 == 0
    scale = math.sqrt(d)
    nq = d // _LANES
    pe_span = PW // CPS + 2

    mesh = plsc.VectorSubcoreMesh(core_axis_name="c", subcore_axis_name="s")

    @functools.partial(
        pl.kernel,
        out_type=jax.ShapeDtypeStruct((seq * ET * BT, 1024), jnp.float32),
        mesh=mesh,
        scratch_types=[
            pltpu.VMEM((PW * NSUB, 128), jnp.int32),
            pltpu.VMEM((K, d), jnp.float32),
            pltpu.VMEM((K, d), jnp.float32),
            pltpu.VMEM((ET, TB, 1024), jnp.float32),
            pltpu.VMEM((pe_span, d), jnp.float32),
            pltpu.SemaphoreType.DMA,
            pltpu.SemaphoreType.DMA,
            pltpu.SemaphoreType.DMA,
        ],
        compiler_params=pltpu.CompilerParams(use_tc_tiling_on_sc=False),
    )
    def emb(idx_hbm, table_hbm, pe_hbm, out_hbm,
            idx_v, rows0_v, rows1_v, tr_v, pe_v,
            g_sem0, g_sem1, o_sem):
        rows_bufs = (rows0_v, rows1_v)
        g_sems = (g_sem0, g_sem1)
        wid = lax.axis_index("s") * nc + lax.axis_index("c")
        base = wid * PW
        spos0 = base // CPS

        def gather_copies(t, b):
            return [
                pltpu.make_async_copy(
                    table_hbm.at[idx_v.at[t * NSUB + jj]],
                    rows_bufs[b].at[pl.ds(jj * 128, 128)],
                    g_sems[b],
                )
                for jj in range(NSUB)
            ]

        def out_copies(t, b):
            g = base + t
            s = g // CPS
            c = g % CPS
            return [
                pltpu.make_async_copy(
                    tr_v.at[et],
                    out_hbm.at[pl.ds((s * ET + et) * BT + TB * c, TB)],
                    o_sem,
                )
                for et in range(ET)
            ]

        riota = jax.lax.iota(jnp.int32, _LANES)
        perms = {st: riota ^ st for st in (8, 4, 2, 1)}
        masks = {st: (riota & st) == 0 for st in (8, 4, 2, 1)}

        def compute(t, b):
            g = base + t
            spos_l = g // CPS - spos0
            rows = rows_bufs[b]
            pe_regs = [pe_v[spos_l, pl.ds(q * _LANES, _LANES)] for q in range(nq)]

            def kb_body(kb, kc):
                b0 = kb * _LANES
                btl = lax.shift_right_logical(kb, 3)
                bi0 = lax.bitwise_and(kb, 7) * _LANES
                for q in range(nq):
                    x = [
                        rows[b0 + i, pl.ds(q * _LANES, _LANES)] * scale + pe_regs[q]
                        for i in range(_LANES)
                    ]
                    for st in (8, 4, 2, 1):
                        y = list(x)
                        for i in range(_LANES):
                            if i & st:
                                continue
                            j = i ^ st
                            u = jnp.where(masks[st], x[j], x[i])
                            up = u.at[perms[st]].get(mode="promise_in_bounds")
                            y[i] = jnp.where(masks[st], x[i], up)
                            y[j] = jnp.where(masks[st], up, x[j])
                        x = y
                    for j in range(_LANES):
                        e = q * _LANES + j
                        tr_v[e // 8, btl, pl.ds((e % 8) * 128 + bi0, _LANES)] = x[j]
                return kc

            lax.fori_loop(0, K // _LANES, kb_body, 0)

        pltpu.sync_copy(pe_hbm.at[pl.ds(spos0, pe_span)], pe_v)
        pltpu.sync_copy(idx_hbm.at[pl.ds(base * NSUB, PW * NSUB)], idx_v)
        for cp in gather_copies(0, 0):
            cp.start()

        def outer(i, carry):
            for b in range(2):
                t = 2 * i + b
                @pl.when(2 * i + b + 1 <= PW - 1)
                def _():
                    for cp in gather_copies(t + 1, 1 - b):
                        cp.start()
                for cp in gather_copies(t, b):
                    cp.wait()
                if b == 1:
                    for cp in out_copies(t - 1, 1 - b):
                        cp.wait()
                else:
                    @pl.when(i >= 1)
                    def _():
                        for cp in out_copies(t - 1, 1 - b):
                            cp.wait()
                compute(t, b)
                for cp in out_copies(t, b):
                    cp.start()
            return carry

        lax.fori_loop(0, PW // 2, outer, 0)
        for cp in out_copies(PW - 1, 1):
            cp.wait()

    return emb


def kernel(batch, embed_weight):
    seq, bsz = batch.shape
    vocab, d = embed_weight.shape
    idx = batch.astype(jnp.int32).reshape(seq * bsz // 128, 128)
    pw = (seq * bsz // 512) // 32
    pe_rows = seq + pw // (bsz // 512) + 2
    pe = jnp.asarray(_pe_table(pe_rows, d))
    z = _build(seq, bsz, vocab, d)(idx, embed_weight, pe)
    z5 = z.reshape(seq, d // 8, bsz // 128, 8, 128)
    out = jnp.transpose(z5, (0, 2, 4, 1, 3)).reshape(seq, bsz, d)
    return out


# speedup vs baseline: 1.1119x; 1.0337x over previous
import functools
import math

import jax
import jax.numpy as jnp
import numpy as np
from jax import lax
from jax.experimental import pallas as pl
from jax.experimental.pallas import tpu as pltpu
from jax.experimental.pallas import tpu_sc as plsc

_LANES = 16


def _pe_table(rows, d):
    position = np.arange(rows, dtype=np.float32)[:, None]
    div_term = np.exp(np.arange(0, d, 2, dtype=np.float32) * (-(math.log(10000.0) / d)))
    pe = np.zeros((rows, d), dtype=np.float32)
    pe[:, 0::2] = np.sin(position * div_term)
    pe[:, 1::2] = np.cos(position * div_term)
    return pe


@functools.lru_cache(maxsize=None)
def _build(seq, batch, vocab, d):
    info = plsc.get_sparse_core_info()
    nc, ns = info.num_cores, info.num_subcores
    nw = nc * ns
    n = seq * batch
    K = 512
    NSUB = K // 128
    TB = K // 128
    ET = d // 8
    BT = batch // 128
    T = n // K
    PW = T // nw
    CPS = batch // K
    assert n % K == 0 and T % nw == 0 and batch % K == 0 and PW % 2 == 0
    scale = math.sqrt(d)
    nq = d // _LANES
    pe_span = PW // CPS + 2

    mesh = plsc.VectorSubcoreMesh(core_axis_name="c", subcore_axis_name="s")

    @functools.partial(
        pl.kernel,
        out_type=jax.ShapeDtypeStruct((seq * ET * BT, 1024), jnp.float32),
        mesh=mesh,
        scratch_types=[
            pltpu.VMEM((PW * NSUB, 128), jnp.int32),
            pltpu.VMEM((K, d), jnp.float32),
            pltpu.VMEM((K, d), jnp.float32),
            pltpu.VMEM((ET, TB, 1024), jnp.float32),
            pltpu.VMEM((pe_span, d), jnp.float32),
            pltpu.SemaphoreType.DMA,
            pltpu.SemaphoreType.DMA,
            pltpu.SemaphoreType.DMA,
        ],
        compiler_params=pltpu.CompilerParams(use_tc_tiling_on_sc=False),
    )
    def emb(idx_hbm, table_hbm, pe_hbm, out_hbm,
            idx_v, rows0_v, rows1_v, tr_v, pe_v,
            g_sem0, g_sem1, o_sem):
        rows_bufs = (rows0_v, rows1_v)
        g_sems = (g_sem0, g_sem1)
        wid = lax.axis_index("s") * nc + lax.axis_index("c")
        base = wid * PW
        spos0 = base // CPS

        def gather_copies(t, b):
            return [
                pltpu.make_async_copy(
                    table_hbm.at[idx_v.at[t * NSUB + jj]],
                    rows_bufs[b].at[pl.ds(jj * 128, 128)],
                    g_sems[b],
                )
                for jj in range(NSUB)
            ]

        def out_copies(t, b):
            g = base + t
            s = g // CPS
            c = g % CPS
            return [
                pltpu.make_async_copy(
                    tr_v.at[et],
                    out_hbm.at[pl.ds((s * ET + et) * BT + TB * c, TB)],
                    o_sem,
                )
                for et in range(ET)
            ]

        riota = jax.lax.iota(jnp.int32, _LANES)
        perms = {st: riota ^ st for st in (8, 4, 2, 1)}
        masks = {st: (riota & st) == 0 for st in (8, 4, 2, 1)}

        def compute(t, b):
            g = base + t
            spos_l = g // CPS - spos0
            rows = rows_bufs[b]
            pe_regs = [pe_v[spos_l, pl.ds(q * _LANES, _LANES)] for q in range(nq)]

            @plsc.parallel_loop(0, K // _LANES, unroll=2)
            def kb_body(kb):
                b0 = kb * _LANES
                btl = lax.shift_right_logical(kb, 3)
                bi0 = lax.bitwise_and(kb, 7) * _LANES
                for q in range(nq):
                    x = [
                        rows[b0 + i, pl.ds(q * _LANES, _LANES)] * scale + pe_regs[q]
                        for i in range(_LANES)
                    ]
                    for st in (8, 4, 2, 1):
                        y = list(x)
                        for i in range(_LANES):
                            if i & st:
                                continue
                            j = i ^ st
                            u = jnp.where(masks[st], x[j], x[i])
                            up = u.at[perms[st]].get(mode="promise_in_bounds")
                            y[i] = jnp.where(masks[st], x[i], up)
                            y[j] = jnp.where(masks[st], up, x[j])
                        x = y
                    for j in range(_LANES):
                        e = q * _LANES + j
                        tr_v[e // 8, btl, pl.ds((e % 8) * 128 + bi0, _LANES)] = x[j]

        pltpu.sync_copy(pe_hbm.at[pl.ds(spos0, pe_span)], pe_v)
        pltpu.sync_copy(idx_hbm.at[pl.ds(base * NSUB, PW * NSUB)], idx_v)
        for cp in gather_copies(0, 0):
            cp.start()

        def outer(i, carry):
            for b in range(2):
                t = 2 * i + b
                @pl.when(2 * i + b + 1 <= PW - 1)
                def _():
                    for cp in gather_copies(t + 1, 1 - b):
                        cp.start()
                for cp in gather_copies(t, b):
                    cp.wait()
                if b == 1:
                    for cp in out_copies(t - 1, 1 - b):
                        cp.wait()
                else:
                    @pl.when(i >= 1)
                    def _():
                        for cp in out_copies(t - 1, 1 - b):
                            cp.wait()
                compute(t, b)
                for cp in out_copies(t, b):
                    cp.start()
            return carry

        lax.fori_loop(0, PW // 2, outer, 0)
        for cp in out_copies(PW - 1, 1):
            cp.wait()

    return emb


def kernel(batch, embed_weight):
    seq, bsz = batch.shape
    vocab, d = embed_weight.shape
    idx = batch.astype(jnp.int32).reshape(seq * bsz // 128, 128)
    pw = (seq * bsz // 512) // 32
    pe_rows = seq + pw // (bsz // 512) + 2
    pe = jnp.asarray(_pe_table(pe_rows, d))
    z = _build(seq, bsz, vocab, d)(idx, embed_weight, pe)
    z5 = z.reshape(seq, d // 8, bsz // 128, 8, 128)
    out = jnp.transpose(z5, (0, 2, 4, 1, 3)).reshape(seq, bsz, d)
    return out
